# Initial kernel scaffold; baseline (speedup 1.0000x reference)
import jax
import jax.numpy as jnp
from jax.experimental import pallas as pl


def kernel(x, W_qkv, W_dw, W_proj, temperature):
    raise NotImplementedError("write your pallas kernel here")


# speedup vs baseline: 1.2492x; 1.2492x over previous
import jax
import jax.numpy as jnp
from jax.experimental import pallas as pl
from jax.experimental.pallas import tpu as pltpu

DIM = 384
HEADS = 8
HD = DIM // HEADS
H = 224
W = 224
NPIX = H * W
TP = 1792
PAD = 256
SLAB = TP + 2 * PAD
NSTEP = NPIX // TP
TOPK = 7

_F32 = jnp.float32
_BF16 = jnp.bfloat16


def _mm_nn(a, b):
    return jax.lax.dot_general(a, b, (((1,), (0,)), ((), ())),
                               preferred_element_type=_F32)


def _mm_nt(a, b):
    return jax.lax.dot_general(a, b, (((1,), (1,)), ((), ())),
                               preferred_element_type=_F32)


def _pass_a(xb_hbm, wqkv, wdw,
            q_ref, k_ref, v_ref, aux_ref,
            xbuf, sem):
    i = pl.program_id(0)

    def start_copy(slot, idx):
        pltpu.make_async_copy(xb_hbm.at[:, pl.ds(idx * TP, SLAB)],
                              xbuf.at[slot], sem.at[slot]).start()

    @pl.when(i == 0)
    def _():
        start_copy(0, 0)

    @pl.when(i + 1 < NSTEP)
    def _():
        start_copy((i + 1) % 2, i + 1)

    slot = i % 2
    pltpu.make_async_copy(xb_hbm.at[:, pl.ds(i * TP, SLAB)],
                          xbuf.at[slot], sem.at[slot]).wait()
    xs = xbuf[slot]

    col = jax.lax.broadcasted_iota(jnp.int32, (1, TP), 1) % W
    m_l = (col > 0).astype(_F32)
    m_r = (col < W - 1).astype(_F32)

    def dwconv(raw_b, w9):
        raw = raw_b.astype(_F32)

        def tap(dr, dc):
            off = PAD + dr * W + dc
            return w9[:, (dr + 1) * 3 + (dc + 1):(dr + 1) * 3 + (dc + 2)] \
                * raw[:, off:off + TP]
        acc_c = tap(-1, 0) + tap(0, 0) + tap(1, 0)
        acc_l = tap(-1, -1) + tap(0, -1) + tap(1, -1)
        acc_r = tap(-1, 1) + tap(0, 1) + tap(1, 1)
        return acc_c + m_l * acc_l + m_r * acc_r

    raw_qk = _mm_nn(wqkv[0:2 * DIM], xs).astype(_BF16)
    q = dwconv(raw_qk[0:DIM], wdw[0:DIM])
    k = dwconv(raw_qk[DIM:2 * DIM], wdw[DIM:2 * DIM])
    q_ref[...] = q
    k_ref[...] = k

    raw_v = _mm_nn(wqkv[2 * DIM:3 * DIM], xs).astype(_BF16)
    v = dwconv(raw_v, wdw[2 * DIM:3 * DIM])
    v_ref[...] = v.astype(_BF16)

    qn2 = jnp.sum(q * q, axis=1, keepdims=True)
    kn2 = jnp.sum(k * k, axis=1, keepdims=True)
    nrm = jnp.concatenate([qn2, kn2, qn2, kn2], axis=1)

    @pl.when(i == 0)
    def _():
        aux_ref[...] = nrm

    @pl.when(i > 0)
    def _():
        aux_ref[...] += nrm


def _pass_a2(q_hbm, k_hbm, aux_ref, g_ref, rr_ref):
    i = pl.program_id(0)

    @pl.when(i == 0)
    def _():
        n = jnp.maximum(jnp.sqrt(aux_ref[...]), 1e-12)
        rr_ref[...] = 1.0 / n

    rq = rr_ref[:, 0:1]
    rk = rr_ref[:, 1:2]
    qn = (q_hbm[...] * rq).astype(_BF16)
    kn = (k_hbm[...] * rk).astype(_BF16)
    g = _mm_nt(qn, kn)

    @pl.when(i == 0)
    def _():
        g_ref[...] = g

    @pl.when(i > 0)
    def _():
        g_ref[...] += g


def _pass_b(g_ref, tcol_ref, wp_ref, v_ref, out_ref, m_ref):
    i = pl.program_id(0)

    @pl.when(i == 0)
    def _():
        s = g_ref[...] * tcol_ref[...]
        rows = jax.lax.broadcasted_iota(jnp.int32, (DIM, DIM), 0) // HD
        cols = jax.lax.broadcasted_iota(jnp.int32, (DIM, DIM), 1) // HD
        neg = _F32(-jnp.inf)
        s = jnp.where(rows == cols, s, neg)
        work = s
        m0 = thr = None
        for t in range(TOPK):
            thr = jnp.max(work, axis=1, keepdims=True)
            if t == 0:
                m0 = thr
            work = jnp.where(work == thr, neg, work)
        p = jnp.where(s >= thr, jnp.exp(s - m0), 0.0)
        b = p / jnp.sum(p, axis=1, keepdims=True)
        m_mat = _mm_nn(wp_ref[...], b.astype(_BF16))
        m_ref[...] = m_mat.astype(_BF16)

    out_ref[...] = _mm_nn(m_ref[...], v_ref[...])


def kernel(x, W_qkv, W_dw, W_proj, temperature):
    xf = x.reshape(DIM, NPIX)
    xp = jnp.zeros((DIM, PAD + NPIX + PAD), _BF16)
    xp = xp.at[:, PAD:PAD + NPIX].set(xf.astype(_BF16))

    wqkv = W_qkv.reshape(3 * DIM, DIM).astype(_BF16)
    wdw = W_dw.reshape(3 * DIM, 9)
    tcol = jnp.repeat(temperature.reshape(HEADS), HD).reshape(DIM, 1)
    wp = W_proj.reshape(DIM, DIM).astype(_BF16)

    q, k, v, aux = pl.pallas_call(
        _pass_a,
        grid=(NSTEP,),
        in_specs=[
            pl.BlockSpec(memory_space=pl.ANY),
            pl.BlockSpec((3 * DIM, DIM), lambda i: (0, 0)),
            pl.BlockSpec((3 * DIM, 9), lambda i: (0, 0)),
        ],
        out_specs=[
            pl.BlockSpec((DIM, TP), lambda i: (0, i)),
            pl.BlockSpec((DIM, TP), lambda i: (0, i)),
            pl.BlockSpec((DIM, TP), lambda i: (0, i)),
            pl.BlockSpec((DIM, 4), lambda i: (0, 0)),
        ],
        out_shape=[
            jax.ShapeDtypeStruct((DIM, NPIX), _F32),
            jax.ShapeDtypeStruct((DIM, NPIX), _F32),
            jax.ShapeDtypeStruct((DIM, NPIX), _BF16),
            jax.ShapeDtypeStruct((DIM, 4), _F32),
        ],
        scratch_shapes=[
            pltpu.VMEM((2, DIM, SLAB), _BF16),
            pltpu.SemaphoreType.DMA((2,)),
        ],
        compiler_params=pltpu.CompilerParams(
            dimension_semantics=("arbitrary",),
        ),
    )(xp, wqkv, wdw)

    g = pl.pallas_call(
        _pass_a2,
        grid=(NSTEP,),
        in_specs=[
            pl.BlockSpec((DIM, TP), lambda i: (0, i)),
            pl.BlockSpec((DIM, TP), lambda i: (0, i)),
            pl.BlockSpec((DIM, 4), lambda i: (0, 0)),
        ],
        out_specs=pl.BlockSpec((DIM, DIM), lambda i: (0, 0)),
        out_shape=jax.ShapeDtypeStruct((DIM, DIM), _F32),
        scratch_shapes=[pltpu.VMEM((DIM, 4), _F32)],
        compiler_params=pltpu.CompilerParams(
            dimension_semantics=("arbitrary",),
        ),
    )(q, k, aux)

    out = pl.pallas_call(
        _pass_b,
        grid=(NSTEP,),
        in_specs=[
            pl.BlockSpec((DIM, DIM), lambda i: (0, 0)),
            pl.BlockSpec((DIM, 1), lambda i: (0, 0)),
            pl.BlockSpec((DIM, DIM), lambda i: (0, 0)),
            pl.BlockSpec((DIM, TP), lambda i: (0, i)),
        ],
        out_specs=pl.BlockSpec((DIM, TP), lambda i: (0, i)),
        out_shape=jax.ShapeDtypeStruct((DIM, NPIX), _F32),
        scratch_shapes=[pltpu.VMEM((DIM, DIM), _BF16)],
        compiler_params=pltpu.CompilerParams(
            dimension_semantics=("arbitrary",),
        ),
    )(g, tcol, wp, v)

    return out.reshape(1, DIM, H, W)


# speedup vs baseline: 2.9642x; 2.3728x over previous
import jax
import jax.numpy as jnp
from jax.experimental import pallas as pl
from jax.experimental.pallas import tpu as pltpu

DIM = 384
HEADS = 8
HD = DIM // HEADS
H = 224
W = 224
NPIX = H * W
TP = 1792
PAD = 256
SLAB = TP + 2 * PAD
NSTEP = NPIX // TP
TOPK = 7

_F32 = jnp.float32
_BF16 = jnp.bfloat16


def _mm_nn(a, b):
    return jax.lax.dot_general(a, b, (((1,), (0,)), ((), ())),
                               preferred_element_type=_F32)


def _mm_tn(a, b):
    return jax.lax.dot_general(a, b, (((0,), (0,)), ((), ())),
                               preferred_element_type=_F32)


def _pass_a(xb_hbm, wqkv, wdw,
            q_ref, k_ref, v_ref, aux_ref,
            xbuf, sem):
    i = pl.program_id(0)

    def start_copy(slot, idx):
        pltpu.make_async_copy(xb_hbm.at[pl.ds(idx * TP, SLAB), :],
                              xbuf.at[slot], sem.at[slot]).start()

    @pl.when(i == 0)
    def _():
        start_copy(0, 0)

    @pl.when(i + 1 < NSTEP)
    def _():
        start_copy((i + 1) % 2, i + 1)

    slot = i % 2
    pltpu.make_async_copy(xb_hbm.at[pl.ds(i * TP, SLAB), :],
                          xbuf.at[slot], sem.at[slot]).wait()
    xs = xbuf[slot]

    col = jax.lax.broadcasted_iota(jnp.int32, (TP, 1), 0) % W
    m_l = (col > 0).astype(_F32)
    m_r = (col < W - 1).astype(_F32)

    def dwconv(raw_b, w9):
        raw = raw_b.astype(_F32)

        def tap(dr, dc):
            off = PAD + dr * W + dc
            return w9[(dr + 1) * 3 + (dc + 1):(dr + 1) * 3 + (dc + 2), :] \
                * raw[off:off + TP, :]
        acc_c = tap(-1, 0) + tap(0, 0) + tap(1, 0)
        acc_l = tap(-1, -1) + tap(0, -1) + tap(1, -1)
        acc_r = tap(-1, 1) + tap(0, 1) + tap(1, 1)
        return acc_c + m_l * acc_l + m_r * acc_r

    q = dwconv(_mm_nn(xs, wqkv[:, 0:DIM]).astype(_BF16),
               wdw[:, 0:DIM])
    k = dwconv(_mm_nn(xs, wqkv[:, DIM:2 * DIM]).astype(_BF16),
               wdw[:, DIM:2 * DIM])
    q_ref[...] = q
    k_ref[...] = k

    v = dwconv(_mm_nn(xs, wqkv[:, 2 * DIM:3 * DIM]).astype(_BF16),
               wdw[:, 2 * DIM:3 * DIM])
    v_ref[...] = v.astype(_BF16)

    qn2 = jnp.sum(q * q, axis=0, keepdims=True)
    kn2 = jnp.sum(k * k, axis=0, keepdims=True)
    nrm = jnp.concatenate([qn2, kn2, qn2, kn2, qn2, kn2, qn2, kn2], axis=0)

    @pl.when(i == 0)
    def _():
        aux_ref[...] = nrm

    @pl.when(i > 0)
    def _():
        aux_ref[...] += nrm


def _pass_a2(q_hbm, k_hbm, aux_ref, g_ref, rr_ref):
    i = pl.program_id(0)

    @pl.when(i == 0)
    def _():
        n = jnp.maximum(jnp.sqrt(aux_ref[...]), 1e-12)
        rr_ref[...] = 1.0 / n

    rq = rr_ref[0:1, :]
    rk = rr_ref[1:2, :]
    qn = (q_hbm[...] * rq).astype(_BF16)
    kn = (k_hbm[...] * rk).astype(_BF16)
    g = _mm_tn(kn, qn)

    @pl.when(i == 0)
    def _():
        g_ref[...] = g

    @pl.when(i > 0)
    def _():
        g_ref[...] += g


def _pass_b(g_ref, trow_ref, wp_ref, v_ref, out_ref, m_ref):
    i = pl.program_id(0)

    @pl.when(i == 0)
    def _():
        s = g_ref[...] * trow_ref[...]
        rows = jax.lax.broadcasted_iota(jnp.int32, (DIM, DIM), 0) // HD
        cols = jax.lax.broadcasted_iota(jnp.int32, (DIM, DIM), 1) // HD
        neg = _F32(-jnp.inf)
        s = jnp.where(rows == cols, s, neg)
        work = s
        m0 = thr = None
        for t in range(TOPK):
            thr = jnp.max(work, axis=0, keepdims=True)
            if t == 0:
                m0 = thr
            work = jnp.where(work == thr, neg, work)
        p = jnp.where(s >= thr, jnp.exp(s - m0), 0.0)
        b = p / jnp.sum(p, axis=0, keepdims=True)
        m_mat = _mm_nn(b.astype(_BF16), wp_ref[...])
        m_ref[...] = m_mat.astype(_BF16)

    out_ref[...] = _mm_nn(v_ref[...], m_ref[...])


def kernel(x, W_qkv, W_dw, W_proj, temperature):
    xt = jnp.transpose(x.reshape(DIM, NPIX)).astype(_BF16)
    xp = jnp.zeros((PAD + NPIX + PAD, DIM), _BF16)
    xp = xp.at[PAD:PAD + NPIX, :].set(xt)

    wqkv = jnp.transpose(W_qkv.reshape(3 * DIM, DIM)).astype(_BF16)
    wdw = jnp.transpose(W_dw.reshape(3 * DIM, 9))
    trow = jnp.repeat(temperature.reshape(HEADS), HD).reshape(1, DIM)
    wp = jnp.transpose(W_proj.reshape(DIM, DIM)).astype(_BF16)

    q, k, v, aux = pl.pallas_call(
        _pass_a,
        grid=(NSTEP,),
        in_specs=[
            pl.BlockSpec(memory_space=pl.ANY),
            pl.BlockSpec((DIM, 3 * DIM), lambda i: (0, 0)),
            pl.BlockSpec((9, 3 * DIM), lambda i: (0, 0)),
        ],
        out_specs=[
            pl.BlockSpec((TP, DIM), lambda i: (i, 0)),
            pl.BlockSpec((TP, DIM), lambda i: (i, 0)),
            pl.BlockSpec((TP, DIM), lambda i: (i, 0)),
            pl.BlockSpec((8, DIM), lambda i: (0, 0)),
        ],
        out_shape=[
            jax.ShapeDtypeStruct((NPIX, DIM), _F32),
            jax.ShapeDtypeStruct((NPIX, DIM), _F32),
            jax.ShapeDtypeStruct((NPIX, DIM), _BF16),
            jax.ShapeDtypeStruct((8, DIM), _F32),
        ],
        scratch_shapes=[
            pltpu.VMEM((2, SLAB, DIM), _BF16),
            pltpu.SemaphoreType.DMA((2,)),
        ],
        compiler_params=pltpu.CompilerParams(
            dimension_semantics=("arbitrary",),
        ),
    )(xp, wqkv, wdw)

    g = pl.pallas_call(
        _pass_a2,
        grid=(NSTEP,),
        in_specs=[
            pl.BlockSpec((TP, DIM), lambda i: (i, 0)),
            pl.BlockSpec((TP, DIM), lambda i: (i, 0)),
            pl.BlockSpec((8, DIM), lambda i: (0, 0)),
        ],
        out_specs=pl.BlockSpec((DIM, DIM), lambda i: (0, 0)),
        out_shape=jax.ShapeDtypeStruct((DIM, DIM), _F32),
        scratch_shapes=[pltpu.VMEM((8, DIM), _F32)],
        compiler_params=pltpu.CompilerParams(
            dimension_semantics=("arbitrary",),
        ),
    )(q, k, aux)

    out = pl.pallas_call(
        _pass_b,
        grid=(NSTEP,),
        in_specs=[
            pl.BlockSpec((DIM, DIM), lambda i: (0, 0)),
            pl.BlockSpec((1, DIM), lambda i: (0, 0)),
            pl.BlockSpec((DIM, DIM), lambda i: (0, 0)),
            pl.BlockSpec((TP, DIM), lambda i: (i, 0)),
        ],
        out_specs=pl.BlockSpec((TP, DIM), lambda i: (i, 0)),
        out_shape=jax.ShapeDtypeStruct((NPIX, DIM), _F32),
        scratch_shapes=[pltpu.VMEM((DIM, DIM), _BF16)],
        compiler_params=pltpu.CompilerParams(
            dimension_semantics=("arbitrary",),
        ),
    )(g, trow, wp, v)

    return jnp.transpose(out).reshape(1, DIM, H, W)


# speedup vs baseline: 3.3606x; 1.1337x over previous
import jax
import jax.numpy as jnp
from jax.experimental import pallas as pl
from jax.experimental.pallas import tpu as pltpu

DIM = 384
HEADS = 8
HD = DIM // HEADS
H = 224
W = 224
NPIX = H * W
TP = 1792
PAD = 256
SLAB = TP + 2 * PAD
NSTEP = NPIX // TP
TOPK = 7

_F32 = jnp.float32
_BF16 = jnp.bfloat16


def _mm_nn(a, b):
    return jax.lax.dot_general(a, b, (((1,), (0,)), ((), ())),
                               preferred_element_type=_F32)


def _mm_tn(a, b):
    return jax.lax.dot_general(a, b, (((0,), (0,)), ((), ())),
                               preferred_element_type=_F32)


def _pass_a(x_hbm, wqkv, wdw,
            q_ref, k_ref, v_ref, aux_ref,
            xbuf, sem):
    i = pl.program_id(0)

    def cp_first(slot):
        return pltpu.make_async_copy(
            x_hbm.at[pl.ds(0, TP + PAD), :],
            xbuf.at[slot, pl.ds(PAD, TP + PAD), :], sem.at[slot])

    def cp_mid(slot, idx):
        return pltpu.make_async_copy(
            x_hbm.at[pl.ds(idx * TP - PAD, SLAB), :],
            xbuf.at[slot], sem.at[slot])

    def cp_last(slot):
        return pltpu.make_async_copy(
            x_hbm.at[pl.ds((NSTEP - 1) * TP - PAD, TP + PAD), :],
            xbuf.at[slot, pl.ds(0, TP + PAD), :], sem.at[slot])

    def start_copy(slot, idx):
        @pl.when(idx < NSTEP - 1)
        def _():
            cp_mid(slot, idx).start()

        @pl.when(idx == NSTEP - 1)
        def _():
            xbuf[slot, TP + PAD:SLAB, :] = jnp.zeros((PAD, DIM), _F32)
            cp_last(slot).start()

    def wait_copy(slot, idx):
        @pl.when(idx == 0)
        def _():
            cp_first(slot).wait()

        @pl.when(jnp.logical_and(idx > 0, idx < NSTEP - 1))
        def _():
            cp_mid(slot, idx).wait()

        @pl.when(idx == NSTEP - 1)
        def _():
            cp_last(slot).wait()

    @pl.when(i == 0)
    def _():
        xbuf[0, 0:PAD, :] = jnp.zeros((PAD, DIM), _F32)
        cp_first(0).start()

    @pl.when(i + 1 < NSTEP)
    def _():
        start_copy((i + 1) % 2, i + 1)

    slot = i % 2
    wait_copy(slot, i)
    xs = xbuf[slot].astype(_BF16)

    col = jax.lax.broadcasted_iota(jnp.int32, (TP, 1), 0) % W
    m_l = (col > 0).astype(_F32)
    m_r = (col < W - 1).astype(_F32)

    def dwconv(raw_b, w9):
        raw = raw_b.astype(_F32)

        def vert(dc_idx, start, n):
            return (w9[dc_idx:dc_idx + 1, :] * raw[start - W:start - W + n, :]
                    + w9[dc_idx + 3:dc_idx + 4, :] * raw[start:start + n, :]
                    + w9[dc_idx + 6:dc_idx + 7, :] * raw[start + W:start + W + n, :])

        vc = vert(1, PAD, TP)
        vl = vert(0, PAD - 8, TP + 8)
        vr = vert(2, PAD, TP + 8)
        return vc + m_l * vl[7:7 + TP, :] + m_r * vr[1:1 + TP, :]

    q = dwconv(_mm_nn(xs, wqkv[:, 0:DIM]).astype(_BF16),
               wdw[:, 0:DIM])
    k = dwconv(_mm_nn(xs, wqkv[:, DIM:2 * DIM]).astype(_BF16),
               wdw[:, DIM:2 * DIM])
    q_ref[...] = q
    k_ref[...] = k

    v = dwconv(_mm_nn(xs, wqkv[:, 2 * DIM:3 * DIM]).astype(_BF16),
               wdw[:, 2 * DIM:3 * DIM])
    v_ref[...] = v.astype(_BF16)

    qn2 = jnp.sum(q * q, axis=0, keepdims=True)
    kn2 = jnp.sum(k * k, axis=0, keepdims=True)
    nrm = jnp.concatenate([qn2, kn2, qn2, kn2, qn2, kn2, qn2, kn2], axis=0)

    @pl.when(i == 0)
    def _():
        aux_ref[...] = nrm

    @pl.when(i > 0)
    def _():
        aux_ref[...] += nrm


def _pass_a2(q_hbm, k_hbm, aux_ref, g_ref, rr_ref):
    i = pl.program_id(0)

    @pl.when(i == 0)
    def _():
        n = jnp.maximum(jnp.sqrt(aux_ref[...]), 1e-12)
        rr_ref[...] = 1.0 / n

    rq = rr_ref[0:1, :]
    rk = rr_ref[1:2, :]
    qn = (q_hbm[...] * rq).astype(_BF16)
    kn = (k_hbm[...] * rk).astype(_BF16)
    g = _mm_tn(kn, qn)

    @pl.when(i == 0)
    def _():
        g_ref[...] = g

    @pl.when(i > 0)
    def _():
        g_ref[...] += g


def _pass_b(g_ref, trow_ref, wp_ref, v_ref, out_ref, m_ref):
    i = pl.program_id(0)

    @pl.when(i == 0)
    def _():
        s = g_ref[...] * trow_ref[...]
        rows = jax.lax.broadcasted_iota(jnp.int32, (DIM, DIM), 0) // HD
        cols = jax.lax.broadcasted_iota(jnp.int32, (DIM, DIM), 1) // HD
        neg = _F32(-jnp.inf)
        s = jnp.where(rows == cols, s, neg)
        work = s
        m0 = thr = None
        for t in range(TOPK):
            thr = jnp.max(work, axis=0, keepdims=True)
            if t == 0:
                m0 = thr
            work = jnp.where(work == thr, neg, work)
        p = jnp.where(s >= thr, jnp.exp(s - m0), 0.0)
        b = p / jnp.sum(p, axis=0, keepdims=True)
        m_mat = _mm_nn(b.astype(_BF16), wp_ref[...])
        m_ref[...] = m_mat.astype(_BF16)

    out_ref[...] = _mm_nn(v_ref[...], m_ref[...])


def kernel(x, W_qkv, W_dw, W_proj, temperature):
    xt = jnp.transpose(x.reshape(DIM, NPIX))

    wqkv = jnp.transpose(W_qkv.reshape(3 * DIM, DIM)).astype(_BF16)
    wdw = jnp.transpose(W_dw.reshape(3 * DIM, 9))
    trow = jnp.repeat(temperature.reshape(HEADS), HD).reshape(1, DIM)
    wp = jnp.transpose(W_proj.reshape(DIM, DIM)).astype(_BF16)

    q, k, v, aux = pl.pallas_call(
        _pass_a,
        grid=(NSTEP,),
        in_specs=[
            pl.BlockSpec(memory_space=pl.ANY),
            pl.BlockSpec((DIM, 3 * DIM), lambda i: (0, 0)),
            pl.BlockSpec((9, 3 * DIM), lambda i: (0, 0)),
        ],
        out_specs=[
            pl.BlockSpec((TP, DIM), lambda i: (i, 0)),
            pl.BlockSpec((TP, DIM), lambda i: (i, 0)),
            pl.BlockSpec((TP, DIM), lambda i: (i, 0)),
            pl.BlockSpec((8, DIM), lambda i: (0, 0)),
        ],
        out_shape=[
            jax.ShapeDtypeStruct((NPIX, DIM), _F32),
            jax.ShapeDtypeStruct((NPIX, DIM), _F32),
            jax.ShapeDtypeStruct((NPIX, DIM), _BF16),
            jax.ShapeDtypeStruct((8, DIM), _F32),
        ],
        scratch_shapes=[
            pltpu.VMEM((2, SLAB, DIM), _F32),
            pltpu.SemaphoreType.DMA((2,)),
        ],
        compiler_params=pltpu.CompilerParams(
            dimension_semantics=("arbitrary",),
        ),
    )(xt, wqkv, wdw)

    g = pl.pallas_call(
        _pass_a2,
        grid=(NSTEP,),
        in_specs=[
            pl.BlockSpec((TP, DIM), lambda i: (i, 0)),
            pl.BlockSpec((TP, DIM), lambda i: (i, 0)),
            pl.BlockSpec((8, DIM), lambda i: (0, 0)),
        ],
        out_specs=pl.BlockSpec((DIM, DIM), lambda i: (0, 0)),
        out_shape=jax.ShapeDtypeStruct((DIM, DIM), _F32),
        scratch_shapes=[pltpu.VMEM((8, DIM), _F32)],
        compiler_params=pltpu.CompilerParams(
            dimension_semantics=("arbitrary",),
        ),
    )(q, k, aux)

    out = pl.pallas_call(
        _pass_b,
        grid=(NSTEP,),
        in_specs=[
            pl.BlockSpec((DIM, DIM), lambda i: (0, 0)),
            pl.BlockSpec((1, DIM), lambda i: (0, 0)),
            pl.BlockSpec((DIM, DIM), lambda i: (0, 0)),
            pl.BlockSpec((TP, DIM), lambda i: (i, 0)),
        ],
        out_specs=pl.BlockSpec((TP, DIM), lambda i: (i, 0)),
        out_shape=jax.ShapeDtypeStruct((NPIX, DIM), _F32),
        scratch_shapes=[pltpu.VMEM((DIM, DIM), _BF16)],
        compiler_params=pltpu.CompilerParams(
            dimension_semantics=("arbitrary",),
        ),
    )(g, trow, wp, v)

    return jnp.transpose(out).reshape(1, DIM, H, W)


# speedup vs baseline: 3.6092x; 1.0740x over previous
import jax
import jax.numpy as jnp
from jax.experimental import pallas as pl
from jax.experimental.pallas import tpu as pltpu

DIM = 384
HEADS = 8
HD = DIM // HEADS
H = 224
W = 224
NPIX = H * W
TP = 1792
PAD = 256
SLAB = TP + 2 * PAD
NSTEP = NPIX // TP
TP2 = 3584
NSTEP2 = NPIX // TP2
TOPK = 7

_F32 = jnp.float32
_BF16 = jnp.bfloat16


def _mm_nn(a, b):
    return jax.lax.dot_general(a, b, (((1,), (0,)), ((), ())),
                               preferred_element_type=_F32)


def _mm_tn(a, b):
    return jax.lax.dot_general(a, b, (((0,), (0,)), ((), ())),
                               preferred_element_type=_F32)


def _mm_nt(a, b):
    return jax.lax.dot_general(a, b, (((1,), (1,)), ((), ())),
                               preferred_element_type=_F32)


def _pass_a(x_hbm, wqkv, wdw,
            q_ref, k_ref, v_ref, aux_ref,
            xbuf, sem):
    i = pl.program_id(0)

    def cp_first(slot):
        return pltpu.make_async_copy(
            x_hbm.at[pl.ds(0, TP + PAD), :],
            xbuf.at[slot, pl.ds(PAD, TP + PAD), :], sem.at[slot])

    def cp_mid(slot, idx):
        return pltpu.make_async_copy(
            x_hbm.at[pl.ds(idx * TP - PAD, SLAB), :],
            xbuf.at[slot], sem.at[slot])

    def cp_last(slot):
        return pltpu.make_async_copy(
            x_hbm.at[pl.ds((NSTEP - 1) * TP - PAD, TP + PAD), :],
            xbuf.at[slot, pl.ds(0, TP + PAD), :], sem.at[slot])

    def start_copy(slot, idx):
        @pl.when(idx < NSTEP - 1)
        def _():
            cp_mid(slot, idx).start()

        @pl.when(idx == NSTEP - 1)
        def _():
            xbuf[slot, TP + PAD:SLAB, :] = jnp.zeros((PAD, DIM), _F32)
            cp_last(slot).start()

    def wait_copy(slot, idx):
        @pl.when(idx == 0)
        def _():
            cp_first(slot).wait()

        @pl.when(jnp.logical_and(idx > 0, idx < NSTEP - 1))
        def _():
            cp_mid(slot, idx).wait()

        @pl.when(idx == NSTEP - 1)
        def _():
            cp_last(slot).wait()

    @pl.when(i == 0)
    def _():
        xbuf[0, 0:PAD, :] = jnp.zeros((PAD, DIM), _F32)
        cp_first(0).start()

    @pl.when(i + 1 < NSTEP)
    def _():
        start_copy((i + 1) % 2, i + 1)

    slot = i % 2
    wait_copy(slot, i)
    xs = xbuf[slot].astype(_BF16)

    col = jax.lax.broadcasted_iota(jnp.int32, (TP, 1), 0) % W
    m_l = (col > 0).astype(_F32)
    m_r = (col < W - 1).astype(_F32)

    def dwconv(raw_b, w9):
        raw = raw_b.astype(_F32)

        def vert(dc_idx, start, n):
            return (w9[dc_idx:dc_idx + 1, :] * raw[start - W:start - W + n, :]
                    + w9[dc_idx + 3:dc_idx + 4, :] * raw[start:start + n, :]
                    + w9[dc_idx + 6:dc_idx + 7, :] * raw[start + W:start + W + n, :])

        vc = vert(1, PAD, TP)
        vl = vert(0, PAD - 8, TP + 8)
        vr = vert(2, PAD, TP + 8)
        return vc + m_l * vl[7:7 + TP, :] + m_r * vr[1:1 + TP, :]

    q = dwconv(_mm_nt(xs, wqkv[0:DIM, :]).astype(_BF16),
               wdw[:, 0:DIM])
    k = dwconv(_mm_nt(xs, wqkv[DIM:2 * DIM, :]).astype(_BF16),
               wdw[:, DIM:2 * DIM])
    q_ref[...] = q
    k_ref[...] = k

    v = dwconv(_mm_nt(xs, wqkv[2 * DIM:3 * DIM, :]).astype(_BF16),
               wdw[:, 2 * DIM:3 * DIM])
    v_ref[...] = v.astype(_BF16)

    qn2 = jnp.sum(q * q, axis=0, keepdims=True)
    kn2 = jnp.sum(k * k, axis=0, keepdims=True)
    nrm = jnp.concatenate([qn2, kn2, qn2, kn2, qn2, kn2, qn2, kn2], axis=0)

    @pl.when(i == 0)
    def _():
        aux_ref[...] = nrm

    @pl.when(i > 0)
    def _():
        aux_ref[...] += nrm


def _pass_bc(q_hbm, k_hbm, aux_ref, trow_ref, wp_ref, v_ref,
             out_ref, rr_ref, g_ref, m_ref):
    i = pl.program_id(0)

    @pl.when(i == 0)
    def _():
        rr_ref[...] = 1.0 / jnp.maximum(jnp.sqrt(aux_ref[...]), 1e-12)

    @pl.when(i < NSTEP2)
    def _():
        qn = (q_hbm[...] * rr_ref[0:1, :]).astype(_BF16)
        kn = (k_hbm[...] * rr_ref[1:2, :]).astype(_BF16)
        g = _mm_tn(kn, qn)

        @pl.when(i == 0)
        def _():
            g_ref[...] = g

        @pl.when(i > 0)
        def _():
            g_ref[...] += g

    @pl.when(i == NSTEP2)
    def _():
        s = g_ref[...] * trow_ref[...]
        rows = jax.lax.broadcasted_iota(jnp.int32, (DIM, DIM), 0) // HD
        cols = jax.lax.broadcasted_iota(jnp.int32, (DIM, DIM), 1) // HD
        neg = _F32(-jnp.inf)
        s = jnp.where(rows == cols, s, neg)
        work = s
        m0 = thr = None
        for t in range(TOPK):
            thr = jnp.max(work, axis=0, keepdims=True)
            if t == 0:
                m0 = thr
            work = jnp.where(work == thr, neg, work)
        p = jnp.where(s >= thr, jnp.exp(s - m0), 0.0)
        b = p / jnp.sum(p, axis=0, keepdims=True)
        m_mat = _mm_nt(b.astype(_BF16), wp_ref[...])
        m_ref[...] = m_mat.astype(_BF16)

    @pl.when(i >= NSTEP2)
    def _():
        out_ref[...] = _mm_nn(v_ref[...], m_ref[...])


def kernel(x, W_qkv, W_dw, W_proj, temperature):
    xt = jnp.transpose(x.reshape(DIM, NPIX))

    wqkv = W_qkv.reshape(3 * DIM, DIM).astype(_BF16)
    wdw = jnp.transpose(W_dw.reshape(3 * DIM, 9))
    trow = jnp.repeat(temperature.reshape(HEADS), HD).reshape(1, DIM)
    wp = W_proj.reshape(DIM, DIM).astype(_BF16)

    q, k, v, aux = pl.pallas_call(
        _pass_a,
        grid=(NSTEP,),
        in_specs=[
            pl.BlockSpec(memory_space=pl.ANY),
            pl.BlockSpec((3 * DIM, DIM), lambda i: (0, 0)),
            pl.BlockSpec((9, 3 * DIM), lambda i: (0, 0)),
        ],
        out_specs=[
            pl.BlockSpec((TP, DIM), lambda i: (i, 0)),
            pl.BlockSpec((TP, DIM), lambda i: (i, 0)),
            pl.BlockSpec((TP, DIM), lambda i: (i, 0)),
            pl.BlockSpec((8, DIM), lambda i: (0, 0)),
        ],
        out_shape=[
            jax.ShapeDtypeStruct((NPIX, DIM), _F32),
            jax.ShapeDtypeStruct((NPIX, DIM), _F32),
            jax.ShapeDtypeStruct((NPIX, DIM), _BF16),
            jax.ShapeDtypeStruct((8, DIM), _F32),
        ],
        scratch_shapes=[
            pltpu.VMEM((2, SLAB, DIM), _F32),
            pltpu.SemaphoreType.DMA((2,)),
        ],
        compiler_params=pltpu.CompilerParams(
            dimension_semantics=("arbitrary",),
        ),
    )(xt, wqkv, wdw)

    out = pl.pallas_call(
        _pass_bc,
        grid=(2 * NSTEP2,),
        in_specs=[
            pl.BlockSpec((TP2, DIM), lambda i: (jnp.minimum(i, NSTEP2 - 1), 0)),
            pl.BlockSpec((TP2, DIM), lambda i: (jnp.minimum(i, NSTEP2 - 1), 0)),
            pl.BlockSpec((8, DIM), lambda i: (0, 0)),
            pl.BlockSpec((1, DIM), lambda i: (0, 0)),
            pl.BlockSpec((DIM, DIM), lambda i: (0, 0)),
            pl.BlockSpec((TP2, DIM), lambda i: (jnp.maximum(i - NSTEP2, 0), 0)),
        ],
        out_specs=pl.BlockSpec((TP2, DIM), lambda i: (jnp.maximum(i - NSTEP2, 0), 0)),
        out_shape=jax.ShapeDtypeStruct((NPIX, DIM), _F32),
        scratch_shapes=[
            pltpu.VMEM((8, DIM), _F32),
            pltpu.VMEM((DIM, DIM), _F32),
            pltpu.VMEM((DIM, DIM), _BF16),
        ],
        compiler_params=pltpu.CompilerParams(
            dimension_semantics=("arbitrary",),
        ),
    )(q, k, aux, trow, wp, v)

    return jnp.transpose(out).reshape(1, DIM, H, W)
